# baseline (speedup 1.0000x reference)
import functools

import jax
import jax.numpy as jnp
from jax import lax
from jax.experimental import pallas as pl
from jax.experimental.pallas import tpu as pltpu
from jax.experimental.pallas import tpu_sc as plsc

B = 16384
D = 128
NC, NS = 2, 16
NW = NC * NS
CH = 128
BLK = 4096

CHUNKS = (12288, 4096)


def _gather_rows(table, eng2d, grk2d, row0, bk):
    rows = bk // CH // NW
    nchunk = 2 * rows
    mesh = plsc.VectorSubcoreMesh(
        core_axis_name="c", subcore_axis_name="s",
        num_cores=NC, num_subcores=NS)

    @functools.partial(
        pl.kernel,
        out_type=jax.ShapeDtypeStruct((2 * bk, D), jnp.float32),
        mesh=mesh,
        scratch_types=(
            [pltpu.VMEM((nchunk, CH), jnp.int32)]
            + [pltpu.VMEM((CH, D), jnp.float32)] * nchunk
            + [pltpu.SemaphoreType.DMA] * (2 * nchunk)
        ),
    )
    def gather_kernel(table_hbm, eng_hbm, grk_hbm, out_hbm, idx_v, *rest):
        bufs = rest[:nchunk]
        gsems = rest[nchunk:2 * nchunk]
        ssems = rest[2 * nchunk:]
        wid = lax.axis_index("s") * NC + lax.axis_index("c")
        wrow = row0 + wid * rows
        for j in range(rows):
            pltpu.sync_copy(eng_hbm.at[pl.ds(wrow + j, 1)],
                            idx_v.at[pl.ds(j, 1)])
            pltpu.sync_copy(grk_hbm.at[pl.ds(wrow + j, 1)],
                            idx_v.at[pl.ds(rows + j, 1)])
        dst = [wid * rows * CH + j * CH for j in range(rows)]
        dst += [bk + wid * rows * CH + j * CH for j in range(rows)]
        gathers = [
            pltpu.async_copy(table_hbm.at[idx_v.at[j]], bufs[j], gsems[j])
            for j in range(nchunk)
        ]
        stores = [None] * nchunk
        for j in range(nchunk):
            gathers[j].wait()
            stores[j] = pltpu.async_copy(
                bufs[j], out_hbm.at[pl.ds(dst[j], CH)], ssems[j])
        for j in range(nchunk):
            stores[j].wait()

    return gather_kernel(table, eng2d, grk2d)


def _mlp_body(eng_ref, grk_ref, w1a_ref, w1b_ref, b1_ref, w2_ref, b2_ref,
              w3_ref, b3_ref, out_ref):
    h = eng_ref[...] @ w1a_ref[...] + grk_ref[...] @ w1b_ref[...] + b1_ref[...]
    h = jnp.maximum(h, 0.0)
    h = jnp.maximum(h @ w2_ref[...] + b2_ref[...], 0.0)
    zt = lax.dot_general(w3_ref[...], h, (((1,), (1,)), ((), ()))) + b3_ref[...]
    out_ref[...] = 1.0 / (1.0 + jnp.exp(-zt))


def _mlp_alias_body(big_ref, eng_ref, grk_ref, w1a_ref, w1b_ref, b1_ref,
                    w2_ref, b2_ref, w3_ref, b3_ref, out_ref):
    del big_ref
    _mlp_body(eng_ref, grk_ref, w1a_ref, w1b_ref, b1_ref, w2_ref, b2_ref,
              w3_ref, b3_ref, out_ref)


def _mlp(emb, bk, blk0, big, W1a, W1b, b1, W2, b2, W3t, b3):
    nblk = bk // BLK
    full = lambda shape: pl.BlockSpec(shape, lambda i: (0, 0))
    in_specs = [
        pl.BlockSpec((BLK, D), lambda i: (i, 0)),
        pl.BlockSpec((BLK, D), lambda i, nblk=nblk: (i + nblk, 0)),
        full((D, D)),
        full((D, D)),
        full((1, D)),
        full((D, 64)),
        full((1, 64)),
        full((1, 64)),
        full((1, 1)),
    ]
    args = (emb, emb, W1a, W1b, b1, W2, b2, W3t, b3)
    body = _mlp_body
    kwargs = {}
    if big is not None:
        in_specs = [pl.BlockSpec(memory_space=pl.ANY)] + in_specs
        args = (big,) + args
        body = _mlp_alias_body
        kwargs = {"input_output_aliases": {0: 0}}
    return pl.pallas_call(
        body,
        grid=(nblk,),
        in_specs=in_specs,
        out_specs=pl.BlockSpec((1, BLK), lambda i, blk0=blk0: (0, i + blk0)),
        out_shape=jax.ShapeDtypeStruct((1, B), jnp.float32),
        **kwargs,
    )(*args)


def kernel(eng_ids, grk_ids, table, W1, b1, W2, b2, W3, b3):
    eng2d = eng_ids.astype(jnp.int32).reshape(B // CH, CH)
    grk2d = grk_ids.astype(jnp.int32).reshape(B // CH, CH)
    W1a, W1b = W1[:D], W1[D:]
    b1r = b1.reshape(1, D)
    b2r = b2.reshape(1, 64)
    W3t = W3.reshape(1, 64)
    b3r = b3.reshape(1, 1)
    out = None
    row0 = 0
    for bk in CHUNKS:
        emb = _gather_rows(table, eng2d, grk2d, row0, bk)
        out = _mlp(emb, bk, row0 * CH // BLK, out,
                   W1a, W1b, b1r, W2, b2r, W3t, b3r)
        row0 += bk // CH
    return out.reshape(B, 1)
